# Initial kernel scaffold; baseline (speedup 1.0000x reference)
import jax
import jax.numpy as jnp
from jax.experimental import pallas as pl


def kernel(x, edge_index, edge_attr, batch_idx, atom_emb, bond_emb, layers, pred1_w, pred1_b, pred2_w, pred2_b):
    raise NotImplementedError("write your pallas kernel here")


# speedup vs baseline: 5.0992x; 5.0992x over previous
import jax
import jax.numpy as jnp
from jax.experimental import pallas as pl
from jax.experimental.pallas import tpu as pltpu

_B = 128
_NPG = 32
_EPG = 64
_N = _B * _NPG
_E = _B * _EPG
_H = 128
_AF = 9
_BF = 3
_V = 128

_G = 16
_NB = _B // _G
_NBLK = _G * _NPG
_EBLK = _G * _EPG
_NEG = -1e9


def _mpnn_body(x_ref, ea_ref, srcc_ref, dstc_ref, srcr_ref, dstr_ref,
               atom_ref, bond_ref, *rest):
    out_ref = rest[-1]
    w = rest[:-1]

    x = x_ref[...]
    ea = ea_ref[...]
    src_c = srcc_ref[...]
    dst_c = dstc_ref[...]
    src_r = srcr_ref[...]
    dst_r = dstr_ref[...]

    f32 = jnp.float32

    iota_nv = jax.lax.broadcasted_iota(jnp.int32, (_NBLK, _V), 1)
    nf = jnp.zeros((_NBLK, _H), f32)
    for i in range(_AF):
        oh = (x[:, i:i + 1] == iota_nv).astype(f32)
        nf = nf + jnp.dot(oh, atom_ref[i], preferred_element_type=f32, precision=jax.lax.Precision.HIGHEST)
    iota_ev = jax.lax.broadcasted_iota(jnp.int32, (_EBLK, _V), 1)
    ef = jnp.zeros((_EBLK, _H), f32)
    for i in range(_BF):
        oh = (ea[:, i:i + 1] == iota_ev).astype(f32)
        ef = ef + jnp.dot(oh, bond_ref[i], preferred_element_type=f32, precision=jax.lax.Precision.HIGHEST)

    e_iota = jax.lax.broadcasted_iota(jnp.int32, (_EBLK, 1), 0)
    gofs = (e_iota // _EPG) * _NPG
    n_iota = jax.lax.broadcasted_iota(jnp.int32, (_EBLK, _NBLK), 1)
    dstoh = ((dst_c + gofs) == n_iota).astype(f32)
    srcoh = ((src_c + gofs) == n_iota).astype(f32)

    key_r = src_r * _NPG + dst_r
    key_c = src_c * _NPG + dst_c
    c_lt_r = (jax.lax.broadcasted_iota(jnp.int32, (_EPG, _EPG), 0) <
              jax.lax.broadcasted_iota(jnp.int32, (_EPG, _EPG), 1))
    lives3 = []
    for gi in range(_G):
        kc = key_c[gi * _EPG:(gi + 1) * _EPG, :]
        kr = key_r[:, gi * _EPG:(gi + 1) * _EPG]
        eq = (kc == kr) & c_lt_r
        dup = jnp.sum(eq.astype(jnp.int32), axis=1, keepdims=True) > 0
        lives3.append(jnp.logical_not(dup).reshape(1, _EPG, 1))

    node_iota3 = jax.lax.broadcasted_iota(jnp.int32, (_NPG, _EPG, _H), 0)

    hidden = jnp.zeros((_NBLK, _H), f32)
    n_layers = (len(w) - 4) // 15
    for li in range(n_layers):
        (m1_w, m1_b, m2_w, m2_b, me_w, me_b, mg_b,
         mlp1_w, mlp1_b, mlp2_w, mlp2_b,
         o1_w, o1_b, o2_w, o2_b) = [r[...] for r in w[li * 15:(li + 1) * 15]]
        z = jnp.concatenate([nf, hidden], axis=-1)
        msg1 = jnp.dot(z, m1_w, preferred_element_type=f32, precision=jax.lax.Precision.HIGHEST) + m1_b
        msg2 = jnp.dot(z, m2_w, preferred_element_type=f32, precision=jax.lax.Precision.HIGHEST) + m2_b
        pre = (jnp.dot(dstoh, msg1, preferred_element_type=f32, precision=jax.lax.Precision.HIGHEST) +
               jnp.dot(srcoh, msg2, preferred_element_type=f32, precision=jax.lax.Precision.HIGHEST) +
               jnp.dot(ef, me_w, preferred_element_type=f32, precision=jax.lax.Precision.HIGHEST) + me_b + mg_b)
        m = jnp.maximum(pre, 0.0)
        m = jnp.maximum(jnp.dot(m, mlp1_w, preferred_element_type=f32, precision=jax.lax.Precision.HIGHEST) + mlp1_b, 0.0)
        m = jnp.dot(m, mlp2_w, preferred_element_type=f32, precision=jax.lax.Precision.HIGHEST) + mlp2_b

        parts = []
        for gi in range(_G):
            m3 = m[gi * _EPG:(gi + 1) * _EPG, :].reshape(1, _EPG, _H)
            d3 = dst_c[gi * _EPG:(gi + 1) * _EPG, :].reshape(1, _EPG, 1)
            sel3 = (node_iota3 == d3) & lives3[gi]
            vals = jnp.where(sel3, m3, _NEG)
            parts.append(jnp.max(vals, axis=1))
        agg = jnp.concatenate(parts, axis=0)

        h1 = jnp.dot(z, o1_w, preferred_element_type=f32, precision=jax.lax.Precision.HIGHEST) + o1_b
        h2 = jnp.dot(agg, o2_w, preferred_element_type=f32, precision=jax.lax.Precision.HIGHEST) + o2_b
        nf = jnp.maximum(h1 + h2, 0.0)
        hidden = nf

    pred1_w, pred1_b, pred2_w, pred2_b = [r[...] for r in w[n_layers * 15:]]
    pool = ((jax.lax.broadcasted_iota(jnp.int32, (_G, _NBLK), 1) // _NPG ==
             jax.lax.broadcasted_iota(jnp.int32, (_G, _NBLK), 0))
            .astype(f32) * (1.0 / _NPG))
    gemb = jnp.dot(pool, nf, preferred_element_type=f32, precision=jax.lax.Precision.HIGHEST)
    hpre = jnp.maximum(jnp.dot(gemb, pred1_w, preferred_element_type=f32, precision=jax.lax.Precision.HIGHEST) + pred1_b, 0.0)
    out_ref[...] = jnp.dot(hpre, pred2_w, preferred_element_type=f32, precision=jax.lax.Precision.HIGHEST) + pred2_b


def kernel(x, edge_index, edge_attr, batch_idx, atom_emb, bond_emb, layers,
           pred1_w, pred1_b, pred2_w, pred2_b):
    del batch_idx
    src_l = (edge_index[0] % _NPG).astype(jnp.int32)
    dst_l = (edge_index[1] % _NPG).astype(jnp.int32)
    src_c = src_l.reshape(_E, 1)
    dst_c = dst_l.reshape(_E, 1)
    src_r = src_l.reshape(1, _E)
    dst_r = dst_l.reshape(1, _E)

    wlist = []
    wspecs = []
    for p in layers:
        for nm in ('m1_w', 'm1_b', 'm2_w', 'm2_b', 'me_w', 'me_b', 'mg_b',
                   'mlp1_w', 'mlp1_b', 'mlp2_w', 'mlp2_b',
                   'o1_w', 'o1_b', 'o2_w', 'o2_b'):
            a = p[nm]
            if a.ndim == 1:
                a = a.reshape(1, -1)
            wlist.append(a)
    wlist += [pred1_w, pred1_b.reshape(1, _H), pred2_w, pred2_b.reshape(1, 1)]
    for a in wlist:
        wspecs.append(pl.BlockSpec(a.shape, lambda i: (0, 0)))

    grid = (_NB,)
    in_specs = [
        pl.BlockSpec((_NBLK, _AF), lambda i: (i, 0)),
        pl.BlockSpec((_EBLK, _BF), lambda i: (i, 0)),
        pl.BlockSpec((_EBLK, 1), lambda i: (i, 0)),
        pl.BlockSpec((_EBLK, 1), lambda i: (i, 0)),
        pl.BlockSpec((1, _EBLK), lambda i: (0, i)),
        pl.BlockSpec((1, _EBLK), lambda i: (0, i)),
        pl.BlockSpec((_AF, _V, _H), lambda i: (0, 0, 0)),
        pl.BlockSpec((_BF, _V, _H), lambda i: (0, 0, 0)),
    ] + wspecs

    out = pl.pallas_call(
        _mpnn_body,
        grid=grid,
        in_specs=in_specs,
        out_specs=pl.BlockSpec((_G, 1), lambda i: (i, 0)),
        out_shape=jax.ShapeDtypeStruct((_B, 1), jnp.float32),
    )(x, edge_attr, src_c, dst_c, src_r, dst_r, atom_emb, bond_emb, *wlist)
    return out


# speedup vs baseline: 6.8344x; 1.3403x over previous
import jax
import jax.numpy as jnp
from jax.experimental import pallas as pl

_B = 128
_NPG = 32
_EPG = 64
_N = _B * _NPG
_E = _B * _EPG
_H = 128
_AF = 9
_BF = 3
_V = 128

_G = 16
_NB = _B // _G
_NBLK = _G * _NPG
_EBLK = _G * _EPG
_NEG = -1e9

_HI = jax.lax.Precision.HIGHEST
_MID = jax.lax.Precision.HIGHEST


def _dot(a, b, prec):
    return jnp.dot(a, b, preferred_element_type=jnp.float32, precision=prec)


def _mpnn_body(x_ref, ea_ref, srcc_ref, dstc_ref, srcr_ref, dstr_ref,
               atom_ref, bond_ref, *rest):
    out_ref = rest[-1]
    w = rest[:-1]

    x = x_ref[...]
    ea = ea_ref[...]
    src_c = srcc_ref[...]
    dst_c = dstc_ref[...]
    src_r = srcr_ref[...]
    dst_r = dstr_ref[...]

    f32 = jnp.float32

    iota_nv = jax.lax.broadcasted_iota(jnp.int32, (_NBLK, _V), 1)
    noh = jnp.concatenate(
        [(x[:, i:i + 1] == iota_nv) for i in range(_AF)], axis=1).astype(f32)
    nf = _dot(noh, atom_ref[...], _MID)
    iota_ev = jax.lax.broadcasted_iota(jnp.int32, (_EBLK, _V), 1)
    eoh = jnp.concatenate(
        [(ea[:, i:i + 1] == iota_ev) for i in range(_BF)], axis=1).astype(f32)
    ef = _dot(eoh, bond_ref[...], _MID)

    e_iota = jax.lax.broadcasted_iota(jnp.int32, (_EBLK, 1), 0)
    gofs = (e_iota // _EPG) * _NPG
    n_iota = jax.lax.broadcasted_iota(jnp.int32, (_EBLK, _NBLK), 1)
    goh = jnp.concatenate([(dst_c + gofs) == n_iota,
                           (src_c + gofs) == n_iota], axis=1).astype(f32)

    key_r = src_r * _NPG + dst_r
    key_c = src_c * _NPG + dst_c
    c_lt_r = (jax.lax.broadcasted_iota(jnp.int32, (_EPG, _EPG), 0) <
              jax.lax.broadcasted_iota(jnp.int32, (_EPG, _EPG), 1))
    live_parts = []
    for gi in range(_G):
        kc = key_c[gi * _EPG:(gi + 1) * _EPG, :]
        kr = key_r[:, gi * _EPG:(gi + 1) * _EPG]
        eq = (kc == kr) & c_lt_r
        live_parts.append(
            jnp.sum(eq.astype(jnp.int32), axis=1, keepdims=True) == 0)
    live_c = jnp.concatenate(live_parts, axis=0)
    dst_m = jnp.where(live_c, dst_c, -1)

    node_iota3 = jax.lax.broadcasted_iota(jnp.int32, (_NPG, _EPG, _H), 0)
    sel3s = [node_iota3 == dst_m[gi * _EPG:(gi + 1) * _EPG, :].reshape(1, _EPG, 1)
             for gi in range(_G)]

    hidden = jnp.zeros((_NBLK, _H), f32)
    n_layers = (len(w) - 4) // 9
    for li in range(n_layers):
        (m12o1_w, me_w, bias_e, mlp1_w, mlp1_b, mlp2_w, mlp2_b, o2_w,
         o12_b) = [r[...] for r in w[li * 9:(li + 1) * 9]]
        z = jnp.concatenate([nf, hidden], axis=-1)
        mm = _dot(z, m12o1_w, _HI)
        msg12 = mm[:, :2 * _H]
        h1 = mm[:, 2 * _H:]
        stacked = jnp.concatenate([msg12[:, :_H], msg12[:, _H:]], axis=0)
        pre = (_dot(goh, stacked, _MID) + _dot(ef, me_w, _HI) + bias_e)
        m = jnp.maximum(pre, 0.0)
        m = jnp.maximum(_dot(m, mlp1_w, _HI) + mlp1_b, 0.0)
        m = _dot(m, mlp2_w, _HI) + mlp2_b

        parts = []
        for gi in range(_G):
            m3 = m[gi * _EPG:(gi + 1) * _EPG, :].reshape(1, _EPG, _H)
            parts.append(jnp.max(jnp.where(sel3s[gi], m3, _NEG), axis=1))
        agg = jnp.concatenate(parts, axis=0)

        h2 = _dot(agg, o2_w, _HI)
        nf = jnp.maximum(h1 + h2 + o12_b, 0.0)
        hidden = nf

    pred1_w, pred1_b, pred2_w, pred2_b = [r[...] for r in w[n_layers * 9:]]
    pool = ((jax.lax.broadcasted_iota(jnp.int32, (_G, _NBLK), 1) // _NPG ==
             jax.lax.broadcasted_iota(jnp.int32, (_G, _NBLK), 0))
            .astype(f32) * (1.0 / _NPG))
    gemb = _dot(pool, nf, _MID)
    hpre = jnp.maximum(_dot(gemb, pred1_w, _HI) + pred1_b, 0.0)
    out_ref[...] = _dot(hpre, pred2_w, _HI) + pred2_b


def kernel(x, edge_index, edge_attr, batch_idx, atom_emb, bond_emb, layers,
           pred1_w, pred1_b, pred2_w, pred2_b):
    del batch_idx
    src_l = (edge_index[0] % _NPG).astype(jnp.int32)
    dst_l = (edge_index[1] % _NPG).astype(jnp.int32)
    src_c = src_l.reshape(_E, 1)
    dst_c = dst_l.reshape(_E, 1)
    src_r = src_l.reshape(1, _E)
    dst_r = dst_l.reshape(1, _E)
    atom_w = atom_emb.reshape(_AF * _V, _H)
    bond_w = bond_emb.reshape(_BF * _V, _H)

    wlist = []
    for p in layers:
        m12o1 = jnp.concatenate([p['m1_w'], p['m2_w'], p['o1_w']], axis=1)
        bias_e = (p['m1_b'] + p['m2_b'] + p['me_b'] + p['mg_b']).reshape(1, _H)
        o12_b = (p['o1_b'] + p['o2_b']).reshape(1, _H)
        wlist += [m12o1, p['me_w'], bias_e,
                  p['mlp1_w'], p['mlp1_b'].reshape(1, _H), p['mlp2_w'],
                  p['mlp2_b'].reshape(1, _H), p['o2_w'], o12_b]
    wlist += [pred1_w, pred1_b.reshape(1, _H), pred2_w, pred2_b.reshape(1, 1)]
    wspecs = [pl.BlockSpec(a.shape, lambda i: (0, 0)) for a in wlist]

    grid = (_NB,)
    in_specs = [
        pl.BlockSpec((_NBLK, _AF), lambda i: (i, 0)),
        pl.BlockSpec((_EBLK, _BF), lambda i: (i, 0)),
        pl.BlockSpec((_EBLK, 1), lambda i: (i, 0)),
        pl.BlockSpec((_EBLK, 1), lambda i: (i, 0)),
        pl.BlockSpec((1, _EBLK), lambda i: (0, i)),
        pl.BlockSpec((1, _EBLK), lambda i: (0, i)),
        pl.BlockSpec(atom_w.shape, lambda i: (0, 0)),
        pl.BlockSpec(bond_w.shape, lambda i: (0, 0)),
    ] + wspecs

    out = pl.pallas_call(
        _mpnn_body,
        grid=grid,
        in_specs=in_specs,
        out_specs=pl.BlockSpec((_G, 1), lambda i: (i, 0)),
        out_shape=jax.ShapeDtypeStruct((_B, 1), jnp.float32),
    )(x, edge_attr, src_c, dst_c, src_r, dst_r, atom_w, bond_w, *wlist)
    return out


# speedup vs baseline: 9.4307x; 1.3799x over previous
import jax
import jax.numpy as jnp
from jax.experimental import pallas as pl

_B = 128
_NPG = 32
_EPG = 64
_N = _B * _NPG
_E = _B * _EPG
_H = 128
_AF = 9
_BF = 3
_V = 128

_G = 16
_NB = _B // _G
_NBLK = _G * _NPG
_EBLK = _G * _EPG
_NEG = -1e9

_HI = jax.lax.Precision.HIGHEST
_LO = jax.lax.Precision.DEFAULT


def _dot(a, b, prec):
    return jnp.dot(a, b, preferred_element_type=jnp.float32, precision=prec)


def _mpnn_body(x_ref, ea_ref, srcc_ref, dstc_ref, srcr_ref, dstr_ref,
               atom_ref, bond_ref, *rest):
    out_ref = rest[-1]
    w = rest[:-1]

    x = x_ref[...]
    ea = ea_ref[...]
    src_c = srcc_ref[...]
    dst_c = dstc_ref[...]
    src_r = srcr_ref[...]
    dst_r = dstr_ref[...]

    f32 = jnp.float32

    iota_nv = jax.lax.broadcasted_iota(jnp.int32, (_NBLK, _V), 1)
    noh = jnp.concatenate(
        [(x[:, i:i + 1] == iota_nv) for i in range(_AF)], axis=1).astype(f32)
    nf = _dot(noh, atom_ref[...], _LO)
    iota_ev = jax.lax.broadcasted_iota(jnp.int32, (_EBLK, _V), 1)
    eoh = jnp.concatenate(
        [(ea[:, i:i + 1] == iota_ev) for i in range(_BF)], axis=1).astype(f32)
    ef = _dot(eoh, bond_ref[...], _LO)

    e_iota = jax.lax.broadcasted_iota(jnp.int32, (_EBLK, 1), 0)
    gofs = (e_iota // _EPG) * _NPG
    n_iota = jax.lax.broadcasted_iota(jnp.int32, (_EBLK, _NBLK), 1)
    goh = jnp.concatenate([(dst_c + gofs) == n_iota,
                           (src_c + gofs) == n_iota], axis=1).astype(f32)

    key_r = src_r * _NPG + dst_r
    key_c = src_c * _NPG + dst_c
    c_lt_r = (jax.lax.broadcasted_iota(jnp.int32, (_EPG, _EPG), 0) <
              jax.lax.broadcasted_iota(jnp.int32, (_EPG, _EPG), 1))
    live_parts = []
    for gi in range(_G):
        kc = key_c[gi * _EPG:(gi + 1) * _EPG, :]
        kr = key_r[:, gi * _EPG:(gi + 1) * _EPG]
        eq = (kc == kr) & c_lt_r
        live_parts.append(
            jnp.sum(eq.astype(jnp.int32), axis=1, keepdims=True) == 0)
    live_c = jnp.concatenate(live_parts, axis=0)
    dst_m = jnp.where(live_c, dst_c, -1)

    node_iota3 = jax.lax.broadcasted_iota(jnp.int32, (_NPG, _EPG, _H), 0)
    sel3s = [node_iota3 == dst_m[gi * _EPG:(gi + 1) * _EPG, :].reshape(1, _EPG, 1)
             for gi in range(_G)]

    hidden = jnp.zeros((_NBLK, _H), f32)
    n_layers = (len(w) - 4) // 9
    for li in range(n_layers):
        (m12o1_w, me_w, bias_e, mlp1_w, mlp1_b, mlp2_w, mlp2_b, o2_w,
         o12_b) = [r[...] for r in w[li * 9:(li + 1) * 9]]
        z = jnp.concatenate([nf, hidden], axis=-1)
        pr = _LO if li == 0 else _HI
        mm = _dot(z, m12o1_w, pr)
        msg12 = mm[:, :2 * _H]
        h1 = mm[:, 2 * _H:]
        stacked = jnp.concatenate([msg12[:, :_H], msg12[:, _H:]], axis=0)
        pre = (_dot(goh, stacked, pr) + _dot(ef, me_w, pr) + bias_e)
        m = jnp.maximum(pre, 0.0)
        m = jnp.maximum(_dot(m, mlp1_w, pr) + mlp1_b, 0.0)
        m = _dot(m, mlp2_w, pr) + mlp2_b

        parts = []
        for gi in range(_G):
            m3 = m[gi * _EPG:(gi + 1) * _EPG, :].reshape(1, _EPG, _H)
            parts.append(jnp.max(jnp.where(sel3s[gi], m3, _NEG), axis=1))
        agg = jnp.concatenate(parts, axis=0)

        h2 = _dot(agg, o2_w, _HI)
        nf = jnp.maximum(h1 + h2 + o12_b, 0.0)
        hidden = nf

    pred1_w, pred1_b, pred2_w, pred2_b = [r[...] for r in w[n_layers * 9:]]
    pool = ((jax.lax.broadcasted_iota(jnp.int32, (_G, _NBLK), 1) // _NPG ==
             jax.lax.broadcasted_iota(jnp.int32, (_G, _NBLK), 0))
            .astype(f32) * (1.0 / _NPG))
    gemb = _dot(pool, nf, _HI)
    hpre = jnp.maximum(_dot(gemb, pred1_w, _HI) + pred1_b, 0.0)
    out_ref[...] = _dot(hpre, pred2_w, _HI) + pred2_b


def kernel(x, edge_index, edge_attr, batch_idx, atom_emb, bond_emb, layers,
           pred1_w, pred1_b, pred2_w, pred2_b):
    del batch_idx
    src_l = (edge_index[0] % _NPG).astype(jnp.int32)
    dst_l = (edge_index[1] % _NPG).astype(jnp.int32)
    src_c = src_l.reshape(_E, 1)
    dst_c = dst_l.reshape(_E, 1)
    src_r = src_l.reshape(1, _E)
    dst_r = dst_l.reshape(1, _E)
    atom_w = atom_emb.reshape(_AF * _V, _H)
    bond_w = bond_emb.reshape(_BF * _V, _H)

    wlist = []
    for p in layers:
        m12o1 = jnp.concatenate([p['m1_w'], p['m2_w'], p['o1_w']], axis=1)
        bias_e = (p['m1_b'] + p['m2_b'] + p['me_b'] + p['mg_b']).reshape(1, _H)
        o12_b = (p['o1_b'] + p['o2_b']).reshape(1, _H)
        wlist += [m12o1, p['me_w'], bias_e,
                  p['mlp1_w'], p['mlp1_b'].reshape(1, _H), p['mlp2_w'],
                  p['mlp2_b'].reshape(1, _H), p['o2_w'], o12_b]
    wlist += [pred1_w, pred1_b.reshape(1, _H), pred2_w, pred2_b.reshape(1, 1)]
    wspecs = [pl.BlockSpec(a.shape, lambda i: (0, 0)) for a in wlist]

    grid = (_NB,)
    in_specs = [
        pl.BlockSpec((_NBLK, _AF), lambda i: (i, 0)),
        pl.BlockSpec((_EBLK, _BF), lambda i: (i, 0)),
        pl.BlockSpec((_EBLK, 1), lambda i: (i, 0)),
        pl.BlockSpec((_EBLK, 1), lambda i: (i, 0)),
        pl.BlockSpec((1, _EBLK), lambda i: (0, i)),
        pl.BlockSpec((1, _EBLK), lambda i: (0, i)),
        pl.BlockSpec(atom_w.shape, lambda i: (0, 0)),
        pl.BlockSpec(bond_w.shape, lambda i: (0, 0)),
    ] + wspecs

    out = pl.pallas_call(
        _mpnn_body,
        grid=grid,
        in_specs=in_specs,
        out_specs=pl.BlockSpec((_G, 1), lambda i: (i, 0)),
        out_shape=jax.ShapeDtypeStruct((_B, 1), jnp.float32),
    )(x, edge_attr, src_c, dst_c, src_r, dst_r, atom_w, bond_w, *wlist)
    return out


# speedup vs baseline: 10.8636x; 1.1519x over previous
import jax
import jax.numpy as jnp
from jax.experimental import pallas as pl

_B = 128
_NPG = 32
_EPG = 64
_N = _B * _NPG
_E = _B * _EPG
_H = 128
_AF = 9
_BF = 3
_V = 128

_G = 16
_NB = _B // _G
_NBLK = _G * _NPG
_EBLK = _G * _EPG
_NEG = -1e9

_HI = jax.lax.Precision.HIGHEST
_LO = jax.lax.Precision.DEFAULT


def _dot(a, b, prec):
    return jnp.dot(a, b, preferred_element_type=jnp.float32, precision=prec)


def _mpnn_body(x_ref, ea_ref, srcc_ref, dstc_ref, srcr_ref, dstr_ref,
               atom_ref, bond_ref, *rest):
    out_ref = rest[-1]
    w = rest[:-1]

    x = x_ref[...]
    ea = ea_ref[...]
    src_c = srcc_ref[...]
    dst_c = dstc_ref[...]
    src_r = srcr_ref[...]
    dst_r = dstr_ref[...]

    f32 = jnp.float32

    iota_nv = jax.lax.broadcasted_iota(jnp.int32, (_NBLK, _V), 1)
    noh = jnp.concatenate(
        [(x[:, i:i + 1] == iota_nv) for i in range(_AF)], axis=1).astype(f32)
    nf = _dot(noh, atom_ref[...], _LO)
    iota_ev = jax.lax.broadcasted_iota(jnp.int32, (_EBLK, _V), 1)
    eoh = jnp.concatenate(
        [(ea[:, i:i + 1] == iota_ev) for i in range(_BF)], axis=1).astype(f32)
    ef = _dot(eoh, bond_ref[...], _LO)

    _GG = 4
    _EG4 = _GG * _EPG
    _NG4 = _GG * _NPG
    e_iota4 = jax.lax.broadcasted_iota(jnp.int32, (_EG4, 1), 0)
    gofs4 = (e_iota4 // _EPG) * _NPG
    n_iota4 = jax.lax.broadcasted_iota(jnp.int32, (_EG4, _NG4), 1)
    gohs = []
    for q in range(_G // _GG):
        dq = dst_c[q * _EG4:(q + 1) * _EG4, :]
        sq = src_c[q * _EG4:(q + 1) * _EG4, :]
        gohs.append(jnp.concatenate([(dq + gofs4) == n_iota4,
                                     (sq + gofs4) == n_iota4],
                                    axis=1).astype(f32))

    key_r = src_r * _NPG + dst_r
    key_c = src_c * _NPG + dst_c
    c_lt_r = (jax.lax.broadcasted_iota(jnp.int32, (_EPG, _EPG), 0) <
              jax.lax.broadcasted_iota(jnp.int32, (_EPG, _EPG), 1))
    live_parts = []
    for gi in range(_G):
        kc = key_c[gi * _EPG:(gi + 1) * _EPG, :]
        kr = key_r[:, gi * _EPG:(gi + 1) * _EPG]
        eq = (kc == kr) & c_lt_r
        live_parts.append(
            jnp.sum(eq.astype(jnp.int32), axis=1, keepdims=True) == 0)
    live_c = jnp.concatenate(live_parts, axis=0)
    dst_m = jnp.where(live_c, dst_c, -1)

    node_iota3 = jax.lax.broadcasted_iota(jnp.int32, (_NPG, _EPG, _H), 0)
    sel3s = [node_iota3 == dst_m[gi * _EPG:(gi + 1) * _EPG, :].reshape(1, _EPG, 1)
             for gi in range(_G)]

    hidden = jnp.zeros((_NBLK, _H), f32)
    n_layers = (len(w) - 4) // 9
    for li in range(n_layers):
        (m12o1_w, me_w, bias_e, mlp1_w, mlp1_b, mlp2_w, mlp2_b, o2_w,
         o12_b) = [r[...] for r in w[li * 9:(li + 1) * 9]]
        z = jnp.concatenate([nf, hidden], axis=-1)
        pr = _LO if li == 0 else _HI
        mm = _dot(z, m12o1_w, pr)
        msg12 = mm[:, :2 * _H]
        h1 = mm[:, 2 * _H:]
        gat_parts = []
        for q in range(_G // _GG):
            stacked_q = jnp.concatenate(
                [msg12[q * _NG4:(q + 1) * _NG4, :_H],
                 msg12[q * _NG4:(q + 1) * _NG4, _H:]], axis=0)
            gat_parts.append(_dot(gohs[q], stacked_q, pr))
        pre = (jnp.concatenate(gat_parts, axis=0)
               + _dot(ef, me_w, pr) + bias_e)
        m = jnp.maximum(pre, 0.0)
        m = jnp.maximum(_dot(m, mlp1_w, pr) + mlp1_b, 0.0)
        m = _dot(m, mlp2_w, pr) + mlp2_b

        parts = []
        for gi in range(_G):
            m3 = m[gi * _EPG:(gi + 1) * _EPG, :].reshape(1, _EPG, _H)
            parts.append(jnp.max(jnp.where(sel3s[gi], m3, _NEG), axis=1))
        agg = jnp.concatenate(parts, axis=0)

        h2 = _dot(agg, o2_w, _HI)
        nf = jnp.maximum(h1 + h2 + o12_b, 0.0)
        hidden = nf

    pred1_w, pred1_b, pred2_w, pred2_b = [r[...] for r in w[n_layers * 9:]]
    pool = ((jax.lax.broadcasted_iota(jnp.int32, (_G, _NBLK), 1) // _NPG ==
             jax.lax.broadcasted_iota(jnp.int32, (_G, _NBLK), 0))
            .astype(f32) * (1.0 / _NPG))
    gemb = _dot(pool, nf, _HI)
    hpre = jnp.maximum(_dot(gemb, pred1_w, _HI) + pred1_b, 0.0)
    out_ref[...] = _dot(hpre, pred2_w, _HI) + pred2_b


def kernel(x, edge_index, edge_attr, batch_idx, atom_emb, bond_emb, layers,
           pred1_w, pred1_b, pred2_w, pred2_b):
    del batch_idx
    src_l = (edge_index[0] % _NPG).astype(jnp.int32)
    dst_l = (edge_index[1] % _NPG).astype(jnp.int32)
    src_c = src_l.reshape(_E, 1)
    dst_c = dst_l.reshape(_E, 1)
    src_r = src_l.reshape(1, _E)
    dst_r = dst_l.reshape(1, _E)
    atom_w = atom_emb.reshape(_AF * _V, _H)
    bond_w = bond_emb.reshape(_BF * _V, _H)

    wlist = []
    for p in layers:
        m12o1 = jnp.concatenate([p['m1_w'], p['m2_w'], p['o1_w']], axis=1)
        bias_e = (p['m1_b'] + p['m2_b'] + p['me_b'] + p['mg_b']).reshape(1, _H)
        o12_b = (p['o1_b'] + p['o2_b']).reshape(1, _H)
        wlist += [m12o1, p['me_w'], bias_e,
                  p['mlp1_w'], p['mlp1_b'].reshape(1, _H), p['mlp2_w'],
                  p['mlp2_b'].reshape(1, _H), p['o2_w'], o12_b]
    wlist += [pred1_w, pred1_b.reshape(1, _H), pred2_w, pred2_b.reshape(1, 1)]
    wspecs = [pl.BlockSpec(a.shape, lambda i: (0, 0)) for a in wlist]

    grid = (_NB,)
    in_specs = [
        pl.BlockSpec((_NBLK, _AF), lambda i: (i, 0)),
        pl.BlockSpec((_EBLK, _BF), lambda i: (i, 0)),
        pl.BlockSpec((_EBLK, 1), lambda i: (i, 0)),
        pl.BlockSpec((_EBLK, 1), lambda i: (i, 0)),
        pl.BlockSpec((1, _EBLK), lambda i: (0, i)),
        pl.BlockSpec((1, _EBLK), lambda i: (0, i)),
        pl.BlockSpec(atom_w.shape, lambda i: (0, 0)),
        pl.BlockSpec(bond_w.shape, lambda i: (0, 0)),
    ] + wspecs

    out = pl.pallas_call(
        _mpnn_body,
        grid=grid,
        in_specs=in_specs,
        out_specs=pl.BlockSpec((_G, 1), lambda i: (i, 0)),
        out_shape=jax.ShapeDtypeStruct((_B, 1), jnp.float32),
    )(x, edge_attr, src_c, dst_c, src_r, dst_r, atom_w, bond_w, *wlist)
    return out


# speedup vs baseline: 12.0527x; 1.1095x over previous
import jax
import jax.numpy as jnp
from jax.experimental import pallas as pl

_B = 128
_NPG = 32
_EPG = 64
_N = _B * _NPG
_E = _B * _EPG
_H = 128
_AF = 9
_BF = 3
_V = 128

_G = 16
_NB = _B // _G
_NBLK = _G * _NPG
_EBLK = _G * _EPG
_NEG = -1e9

_HI = jax.lax.Precision.HIGHEST
_LO = jax.lax.Precision.DEFAULT


def _dot(a, b, prec):
    return jnp.dot(a, b, preferred_element_type=jnp.float32, precision=prec)


def _mpnn_body(x_ref, ea_ref, srcc_ref, dstc_ref, srcr_ref, dstr_ref,
               atom_ref, bond_ref, *rest):
    out_ref = rest[-1]
    w = rest[:-1]

    x = x_ref[...]
    ea = ea_ref[...]
    src_c = srcc_ref[...]
    dst_c = dstc_ref[...]
    src_r = srcr_ref[...]
    dst_r = dstr_ref[...]

    f32 = jnp.float32

    iota_nv = jax.lax.broadcasted_iota(jnp.int32, (_NBLK, _V), 1)
    noh = jnp.concatenate(
        [(x[:, i:i + 1] == iota_nv) for i in range(_AF)], axis=1).astype(f32)
    nf = _dot(noh, atom_ref[...], _LO)
    iota_ev = jax.lax.broadcasted_iota(jnp.int32, (_EBLK, _V), 1)
    eoh = jnp.concatenate(
        [(ea[:, i:i + 1] == iota_ev) for i in range(_BF)], axis=1).astype(f32)
    ef = _dot(eoh, bond_ref[...], _LO)

    _GG = 4
    _EG4 = _GG * _EPG
    _NG4 = _GG * _NPG
    e_iota4 = jax.lax.broadcasted_iota(jnp.int32, (_EG4, 1), 0)
    gofs4 = (e_iota4 // _EPG) * _NPG
    n_iota4 = jax.lax.broadcasted_iota(jnp.int32, (_EG4, _NG4), 1)
    gohs = []
    for q in range(_G // _GG):
        dq = dst_c[q * _EG4:(q + 1) * _EG4, :]
        sq = src_c[q * _EG4:(q + 1) * _EG4, :]
        gohs.append(jnp.concatenate([(dq + gofs4) == n_iota4,
                                     (sq + gofs4) == n_iota4],
                                    axis=1).astype(f32))

    key_r = src_r * _NPG + dst_r
    key_c = src_c * _NPG + dst_c
    c_lt_r = (jax.lax.broadcasted_iota(jnp.int32, (_EPG, _EPG), 0) <
              jax.lax.broadcasted_iota(jnp.int32, (_EPG, _EPG), 1))
    live_parts = []
    for gi in range(_G):
        kc = key_c[gi * _EPG:(gi + 1) * _EPG, :]
        kr = key_r[:, gi * _EPG:(gi + 1) * _EPG]
        eq = (kc == kr) & c_lt_r
        live_parts.append(
            jnp.sum(eq.astype(jnp.int32), axis=1, keepdims=True) == 0)
    live_c = jnp.concatenate(live_parts, axis=0)
    dst_m = jnp.where(live_c, dst_c, -1)

    node_iota3 = jax.lax.broadcasted_iota(jnp.int32, (_NPG, _EPG, _H), 0)
    sel3s = [node_iota3 == dst_m[gi * _EPG:(gi + 1) * _EPG, :].reshape(1, _EPG, 1)
             for gi in range(_G)]

    hidden = jnp.zeros((_NBLK, _H), f32)
    n_layers = (len(w) - 4) // 13
    for li in range(n_layers):
        (m12o1_w, me_w, mlp1_w, mlp2_w, o2_w, m1_b, m2_b, me_b, mg_b,
         o1_b, o2_b, mlp1_b, mlp2_b) = [r[...] for r in
                                        w[li * 13:(li + 1) * 13]]
        z = jnp.concatenate([nf, hidden], axis=-1)
        zpr = _LO if li == 0 else _HI
        mm = _dot(z, m12o1_w, zpr)
        msg1 = mm[:, :_H] + m1_b
        msg2 = mm[:, _H:2 * _H] + m2_b
        h1 = mm[:, 2 * _H:] + o1_b
        gat_parts = []
        for q in range(_G // _GG):
            stacked_q = jnp.concatenate(
                [msg1[q * _NG4:(q + 1) * _NG4, :],
                 msg2[q * _NG4:(q + 1) * _NG4, :]], axis=0)
            gat_parts.append(_dot(gohs[q], stacked_q, zpr))
        pre = jnp.concatenate(gat_parts, axis=0)
        pre = pre + (_dot(ef, me_w, _LO) + me_b)
        pre = pre + mg_b
        m = jnp.maximum(pre, 0.0)
        m = jnp.maximum(_dot(m, mlp1_w, _LO) + mlp1_b, 0.0)
        m = _dot(m, mlp2_w, _LO) + mlp2_b

        parts = []
        for gi in range(_G):
            m3 = m[gi * _EPG:(gi + 1) * _EPG, :].reshape(1, _EPG, _H)
            parts.append(jnp.max(jnp.where(sel3s[gi], m3, _NEG), axis=1))
        agg = jnp.concatenate(parts, axis=0)

        h2 = _dot(agg, o2_w, _HI) + o2_b
        nf = jnp.maximum(h1 + h2, 0.0)
        hidden = nf

    pred1_w, pred1_b, pred2_w, pred2_b = [r[...] for r in w[n_layers * 13:]]
    pool = ((jax.lax.broadcasted_iota(jnp.int32, (_G, _NBLK), 1) // _NPG ==
             jax.lax.broadcasted_iota(jnp.int32, (_G, _NBLK), 0))
            .astype(f32) * (1.0 / _NPG))
    gemb = _dot(pool, nf, _HI)
    hpre = jnp.maximum(_dot(gemb, pred1_w, _LO) + pred1_b, 0.0)
    out_ref[...] = _dot(hpre, pred2_w, _LO) + pred2_b


def kernel(x, edge_index, edge_attr, batch_idx, atom_emb, bond_emb, layers,
           pred1_w, pred1_b, pred2_w, pred2_b):
    del batch_idx
    src_l = (edge_index[0] % _NPG).astype(jnp.int32)
    dst_l = (edge_index[1] % _NPG).astype(jnp.int32)
    src_c = src_l.reshape(_E, 1)
    dst_c = dst_l.reshape(_E, 1)
    src_r = src_l.reshape(1, _E)
    dst_r = dst_l.reshape(1, _E)
    atom_w = atom_emb.reshape(_AF * _V, _H)
    bond_w = bond_emb.reshape(_BF * _V, _H)

    def _rb(a):
        return a.astype(jnp.bfloat16).astype(jnp.float32)

    wlist = []
    for p in layers:
        m12o1 = _rb(jnp.concatenate([p['m1_w'], p['m2_w'], p['o1_w']], axis=1))
        wlist += [m12o1, p['me_w'], p['mlp1_w'], p['mlp2_w'], _rb(p['o2_w'])]
        wlist += [p[k].reshape(1, _H) for k in
                  ('m1_b', 'm2_b', 'me_b', 'mg_b', 'o1_b', 'o2_b',
                   'mlp1_b', 'mlp2_b')]
    wlist += [pred1_w, pred1_b.reshape(1, _H), pred2_w, pred2_b.reshape(1, 1)]
    wspecs = [pl.BlockSpec(a.shape, lambda i: (0, 0)) for a in wlist]

    grid = (_NB,)
    in_specs = [
        pl.BlockSpec((_NBLK, _AF), lambda i: (i, 0)),
        pl.BlockSpec((_EBLK, _BF), lambda i: (i, 0)),
        pl.BlockSpec((_EBLK, 1), lambda i: (i, 0)),
        pl.BlockSpec((_EBLK, 1), lambda i: (i, 0)),
        pl.BlockSpec((1, _EBLK), lambda i: (0, i)),
        pl.BlockSpec((1, _EBLK), lambda i: (0, i)),
        pl.BlockSpec(atom_w.shape, lambda i: (0, 0)),
        pl.BlockSpec(bond_w.shape, lambda i: (0, 0)),
    ] + wspecs

    out = pl.pallas_call(
        _mpnn_body,
        grid=grid,
        in_specs=in_specs,
        out_specs=pl.BlockSpec((_G, 1), lambda i: (i, 0)),
        out_shape=jax.ShapeDtypeStruct((_B, 1), jnp.float32),
    )(x, edge_attr, src_c, dst_c, src_r, dst_r, atom_w, bond_w, *wlist)
    return out
